# Initial kernel scaffold; baseline (speedup 1.0000x reference)
import jax
import jax.numpy as jnp
from jax.experimental import pallas as pl


def kernel(x, edge_index, batch, edge_attr, pos, n1_w1, n1_w2, n1_b2, pool1_w, n2_w1, n2_w2, n2_b2, pool2_w, fc1_w, fc1_b, bn1_g, bn1_b, fc2_w, fc2_b, bn2_g, bn2_b, heads_w, heads_b):
    raise NotImplementedError("write your pallas kernel here")


# speedup vs baseline: 6.6963x; 6.6963x over previous
import functools

import jax
import jax.numpy as jnp
from jax import lax
from jax.experimental import pallas as pl
from jax.experimental.pallas import tpu as pltpu
from jax.experimental.pallas import tpu_sc as plsc

N = 800
B = 4
R = 200
E = 32000
INDIM = 200
D1 = 32
D2 = 32
K = 8
NCLASS = 2

_NW = 32
_EPW = E // _NW
_EPAD = 1024
_ROWS = N // 16
_FLAT = N * N


def _sc_body(src_hbm, dst_hbm, attr_hbm, a_out, c_out,
             srcv, dstv, attrv, aidx, aval, cidx, cval, zbuf, vbounce,
             a_sh, c_sh):
    c = lax.axis_index("c")
    s = lax.axis_index("s")
    wid = c * 16 + s

    def _z(i, _):
        zbuf[pl.ds(pl.multiple_of(i * 16, 8), 16)] = jnp.zeros((16,), jnp.float32)
        return 0
    lax.fori_loop(0, 500, _z, 0)
    for kk in range(5):
        pltpu.sync_copy(zbuf.at[pl.ds(0, 8000)],
                        a_sh.at[pl.ds(pl.multiple_of(s * 40000 + kk * 8000, 8), 8000)])

    @pl.when(s == 0)
    def _zc():
        pltpu.sync_copy(zbuf.at[pl.ds(0, 800)], c_sh.at[pl.ds(0, 800)])

    off = pl.multiple_of(wid * _EPW, 8)
    pltpu.sync_copy(src_hbm.at[pl.ds(off, _EPAD)], srcv)
    pltpu.sync_copy(dst_hbm.at[pl.ds(off, _EPAD)], dstv)
    pltpu.sync_copy(attr_hbm.at[pl.ds(off, _EPAD)], attrv)

    for i in range(_EPAD // 16):
        sl = pl.ds(i * 16, 16)
        sv = srcv[sl]
        dv = dstv[sl]
        wv = attrv[sl]
        m = (lax.iota(jnp.int32, 16) + i * 16) < _EPW
        fi = jnp.where(m, sv * N + dv, 0)
        wv = jnp.where(m, wv, 0.0)
        r = i // 8
        cb = pl.ds((i % 8) * 16, 16)
        aidx[r, cb] = fi
        aval[r, cb] = wv
        cidx[r, cb] = jnp.where(m, dv, 0)
        cval[r, cb] = jnp.where(m, 1.0, 0.0)

    plsc.subcore_barrier()
    for j in range(8):
        pltpu.sync_copy(aval.at[j], a_sh.at[aidx.at[j]], add=True)
        pltpu.sync_copy(cval.at[j], c_sh.at[cidx.at[j]], add=True)
    plsc.subcore_barrier()

    dst_off = pl.multiple_of(c * _FLAT + s * 40000, 8)
    pltpu.sync_copy(a_sh.at[pl.ds(pl.multiple_of(s * 40000, 8), 40000)], vbounce)
    pltpu.sync_copy(vbounce, a_out.at[pl.ds(dst_off, 40000)])

    @pl.when(s == 1)
    def _wc():
        pltpu.sync_copy(c_sh.at[pl.ds(0, 800)], zbuf.at[pl.ds(0, 800)])
        pltpu.sync_copy(zbuf.at[pl.ds(0, 800)],
                        c_out.at[pl.ds(pl.multiple_of(c * N, 8), 800)])


@jax.jit
def _sc_scatter(srcp, dstp, attrp):
    f32 = jnp.float32
    kfn = pl.kernel(
        _sc_body,
        mesh=plsc.VectorSubcoreMesh(core_axis_name="c", subcore_axis_name="s"),
        out_type=[jax.ShapeDtypeStruct((2 * _FLAT,), f32),
                  jax.ShapeDtypeStruct((2 * N,), f32)],
        scratch_types=[
            pltpu.VMEM((_EPAD,), jnp.int32),
            pltpu.VMEM((_EPAD,), jnp.int32),
            pltpu.VMEM((_EPAD,), f32),
            pltpu.VMEM((8, 128), jnp.int32),
            pltpu.VMEM((8, 128), f32),
            pltpu.VMEM((8, 128), jnp.int32),
            pltpu.VMEM((8, 128), f32),
            pltpu.VMEM((8000,), f32),
            pltpu.VMEM((40000,), f32),
            pltpu.VMEM_SHARED((_FLAT,), f32),
            pltpu.VMEM_SHARED((N,), f32),
        ],
    )
    return kfn(srcp, dstp, attrp)


def _dot(a, b, ca, cb):
    return lax.dot_general(a, b, (((ca,), (cb,)), ((), ())),
                           precision=lax.Precision.HIGHEST,
                           preferred_element_type=jnp.float32)


def _bdot(a, b, ca, cb):
    return lax.dot_general(a.astype(jnp.bfloat16), b.astype(jnp.bfloat16),
                           (((ca,), (cb,)), ((), ())),
                           preferred_element_type=jnp.float32)


def _bf(x):
    return x.astype(jnp.bfloat16).astype(jnp.float32)


def _sigmoid(x):
    return 1.0 / (1.0 + jnp.exp(-x))


def _tc_body(x_ref, pos_ref, a0a_ref, a0b_ref, cnt0_ref, cnt1_ref,
             w1cat_ref, n1w1_ref, p1w_ref, w2cat_ref, n2w1_ref, p2w_ref,
             fc1w_ref, fc1b_ref, bn1g_ref, bn1b_ref,
             fc2w_ref, fc2b_ref, bn2g_ref, bn2b_ref,
             hw0_ref, hw1_ref, hb_ref,
             out_ref, s1o_ref, s2o_ref, p1o_ref, p2o_ref):
    f32 = jnp.float32
    x = x_ref[...]
    pos = pos_ref[...]
    A0 = a0a_ref[...] + a0b_ref[...]
    cnt_row = cnt0_ref[...] + cnt1_ref[...]

    i0 = lax.broadcasted_iota(jnp.int32, (N, N), 0)
    i1 = lax.broadcasted_iota(jnp.int32, (N, N), 1)
    eye = jnp.where(i0 == i1, 1.0, 0.0).astype(f32)
    same = (i0 // R) == (i1 // R)
    arange_col = lax.broadcasted_iota(jnp.int32, (N, 1), 0).astype(f32)
    i0f = i0.astype(f32)
    goff_row = ((lax.broadcasted_iota(jnp.int32, (1, N), 1) // R) * R).astype(f32)

    def col_of(row):
        return jnp.transpose(row, (1, 0))

    def row_of(col):
        return jnp.transpose(col, (1, 0))

    h = _bf(jnp.maximum(_bdot(pos, n1w1_ref[...], 1, 0), 0.0))
    G = _dot(x, w1cat_ref[...], 1, 0)
    xt = G[:, K * D1:(K + 1) * D1]
    for k in range(K):
        xt = xt + h[:, k:k + 1] * G[:, k * D1:(k + 1) * D1]

    agg = _dot(A0, xt, 0, 0)
    cnt_col = col_of(jnp.maximum(cnt_row, 1.0))
    h1 = agg / cnt_col

    p1w = p1w_ref[...]
    nrm1 = jnp.sqrt(jnp.sum(p1w * p1w, keepdims=True))
    s1_col = _sigmoid(_bdot(h1, p1w, 1, 0) / nrm1)
    s1_row = row_of(s1_col)

    Dm = same & ((s1_col > s1_row) | ((s1_col == s1_row) & (i0 < i1)))
    rank1_row = jnp.sum(jnp.where(Dm, 1.0, 0.0), axis=0, keepdims=True)
    inv1_row = rank1_row + goff_row
    inv1_col = col_of(inv1_row)
    P1 = jnp.where(inv1_row == i0f, 1.0, 0.0)
    perm1_col = _dot(P1, arange_col, 1, 0)
    s1_sorted = _dot(P1, s1_col, 1, 0)
    p1o_ref[...] = perm1_col.astype(jnp.int32)
    s1o_ref[...] = _sigmoid(s1_sorted)

    h1s = h1 * s1_col
    xp = _dot(P1, h1s, 1, 0)
    posp = _dot(P1, pos, 1, 0)

    parts = []
    for g in range(B):
        blk = xp[g * R:(g + 1) * R]
        parts.append(jnp.concatenate(
            [jnp.max(blk, axis=0, keepdims=True),
             jnp.sum(blk, axis=0, keepdims=True) / float(R)], axis=1))
    x1 = jnp.concatenate(parts, axis=0)

    Ap = _dot(_dot(P1, A0, 1, 0), P1, 1, 1)
    Bp = Ap + eye
    M0 = _bdot(Bp, Bp, 1, 0)
    M0 = M0 * (1.0 - eye)

    h2m = _bf(jnp.maximum(_bdot(posp, n2w1_ref[...], 1, 0), 0.0))
    G2 = _dot(xp, w2cat_ref[...], 1, 0)
    xt2 = G2[:, K * D2:(K + 1) * D2]
    for k in range(K):
        xt2 = xt2 + h2m[:, k:k + 1] * G2[:, k * D2:(k + 1) * D2]

    nz_row = jnp.sum(jnp.where(M0 != 0.0, 1.0, 0.0), axis=0, keepdims=True)
    cnt2_col = col_of(jnp.maximum(nz_row, 1.0))
    h2 = _bdot(M0, xt2, 0, 0) / cnt2_col

    p2w = p2w_ref[...]
    nrm2 = jnp.sqrt(jnp.sum(p2w * p2w, keepdims=True))
    s2_col = _sigmoid(_bdot(h2, p2w, 1, 0) / nrm2)
    s2_row = row_of(s2_col)

    D2m = same & ((s2_col > s2_row) | ((s2_col == s2_row) & (i0 < i1)))
    rank2_row = jnp.sum(jnp.where(D2m, 1.0, 0.0), axis=0, keepdims=True)
    rank2_row = rank2_row + goff_row
    P2 = jnp.where(rank2_row == i0f, 1.0, 0.0)
    perm2_col = _dot(P2, arange_col, 1, 0)
    s2_sorted = _dot(P2, s2_col, 1, 0)
    p2o_ref[...] = perm2_col.astype(jnp.int32)
    s2o_ref[...] = _sigmoid(s2_sorted)

    h2s = h2 * s2_col
    parts = []
    for g in range(B):
        blk = h2s[g * R:(g + 1) * R]
        parts.append(jnp.concatenate(
            [jnp.max(blk, axis=0, keepdims=True),
             jnp.sum(blk, axis=0, keepdims=True) / float(R)], axis=1))
    x2 = jnp.concatenate(parts, axis=0)

    xc = jnp.concatenate([x1, x2], axis=1)
    hh = jnp.maximum(_bdot(xc, fc1w_ref[...], 1, 0) + fc1b_ref[...], 0.0)

    def bn(v, g_, b_):
        m = jnp.sum(v, axis=0, keepdims=True) / float(B)
        d = v - m
        var = jnp.sum(d * d, axis=0, keepdims=True) / float(B)
        return d / jnp.sqrt(var + 1e-5) * g_ + b_

    hh = bn(hh, bn1g_ref[...], bn1b_ref[...])
    z = _bdot(hh, fc2w_ref[...], 1, 0) + fc2b_ref[...]
    zm = z - jnp.max(z, axis=1, keepdims=True)
    ls = zm - jnp.log(jnp.sum(jnp.exp(zm), axis=1, keepdims=True))
    hh = bn(ls, bn2g_ref[...], bn2b_ref[...])

    hb = hb_ref[...]
    out_ref[0:B, :] = _bdot(hh, hw0_ref[...], 1, 0) + hb[0:1, :]
    out_ref[B:2 * B, :] = _bdot(hh, hw1_ref[...], 1, 0) + hb[1:2, :]


@functools.partial(jax.jit, static_argnames=("interpret",))
def _tc_main(args, interpret=False):
    f32 = jnp.float32
    outs = [
        jax.ShapeDtypeStruct((2 * B, D2), f32),
        jax.ShapeDtypeStruct((N, 1), f32),
        jax.ShapeDtypeStruct((N, 1), f32),
        jax.ShapeDtypeStruct((N, 1), jnp.int32),
        jax.ShapeDtypeStruct((N, 1), jnp.int32),
    ]
    return pl.pallas_call(_tc_body, out_shape=outs, interpret=interpret)(*args)


def kernel(x, edge_index, batch, edge_attr, pos, n1_w1, n1_w2, n1_b2, pool1_w,
           n2_w1, n2_w2, n2_b2, pool2_w, fc1_w, fc1_b, bn1_g, bn1_b,
           fc2_w, fc2_b, bn2_g, bn2_b, heads_w, heads_b):
    f32 = jnp.float32
    pad = _NW * _EPAD - E
    srcp = jnp.concatenate([edge_index[0], jnp.zeros((pad,), jnp.int32)])
    dstp = jnp.concatenate([edge_index[1], jnp.zeros((pad,), jnp.int32)])
    attrp = jnp.concatenate([edge_attr, jnp.zeros((pad,), f32)])

    a_flat, c_flat = _sc_scatter(srcp, dstp, attrp)
    a0a = a_flat[:_FLAT].reshape(N, N)
    a0b = a_flat[_FLAT:].reshape(N, N)
    cnt0 = c_flat[:N].reshape(1, N)
    cnt1 = c_flat[N:].reshape(1, N)

    rbf = lambda a: a.astype(jnp.bfloat16).astype(f32)
    w1cat = jnp.concatenate(
        [rbf(n1_w2.reshape(K, INDIM, D1).transpose(1, 0, 2).reshape(INDIM, K * D1)),
         n1_b2.reshape(INDIM, D1)], axis=1)
    w2cat = jnp.concatenate(
        [rbf(n2_w2.reshape(K, D1, D2).transpose(1, 0, 2).reshape(D1, K * D2)),
         n2_b2.reshape(D1, D2)], axis=1)

    args = (x, pos, a0a, a0b, cnt0, cnt1,
            w1cat, n1_w1, pool1_w.reshape(D1, 1), w2cat, n2_w1,
            pool2_w.reshape(D2, 1),
            fc1_w, fc1_b.reshape(1, D2), bn1_g.reshape(1, D2),
            bn1_b.reshape(1, D2),
            fc2_w, fc2_b.reshape(1, NCLASS), bn2_g.reshape(1, NCLASS),
            bn2_b.reshape(1, NCLASS),
            heads_w[0], heads_w[1], heads_b)

    out_s, s1o, s2o, p1o, p2o = _tc_main(args)
    outputs = out_s.reshape(2, B, D2)
    return (outputs, pool1_w, pool2_w,
            s1o.reshape(B, R), s2o.reshape(B, R),
            p1o.reshape(N), p2o.reshape(N))


# speedup vs baseline: 6.7004x; 1.0006x over previous
import functools

import jax
import jax.numpy as jnp
from jax import lax
from jax.experimental import pallas as pl
from jax.experimental.pallas import tpu as pltpu
from jax.experimental.pallas import tpu_sc as plsc

N = 800
B = 4
R = 200
E = 32000
INDIM = 200
D1 = 32
D2 = 32
K = 8
NCLASS = 2

_NW = 32
_EPW = E // _NW
_EPAD = 1024
_ROWS = N // 16
_FLAT = N * N


def _sc_body(src_hbm, dst_hbm, attr_hbm, a_out, c_out,
             srcv, dstv, attrv, aidx, aval, cidx, cval, zbuf, vbounce,
             a_sh, c_sh):
    c = lax.axis_index("c")
    s = lax.axis_index("s")
    wid = c * 16 + s

    def _z(i, _):
        zbuf[pl.ds(pl.multiple_of(i * 16, 8), 16)] = jnp.zeros((16,), jnp.float32)
        return 0
    lax.fori_loop(0, 500, _z, 0)
    for kk in range(5):
        pltpu.sync_copy(zbuf.at[pl.ds(0, 8000)],
                        a_sh.at[pl.ds(pl.multiple_of(s * 40000 + kk * 8000, 8), 8000)])

    @pl.when(s == 0)
    def _zc():
        pltpu.sync_copy(zbuf.at[pl.ds(0, 800)], c_sh.at[pl.ds(0, 800)])

    off = pl.multiple_of(wid * _EPW, 8)
    pltpu.sync_copy(src_hbm.at[pl.ds(off, _EPAD)], srcv)
    pltpu.sync_copy(dst_hbm.at[pl.ds(off, _EPAD)], dstv)
    pltpu.sync_copy(attr_hbm.at[pl.ds(off, _EPAD)], attrv)

    for i in range(_EPAD // 16):
        sl = pl.ds(i * 16, 16)
        sv = srcv[sl]
        dv = dstv[sl]
        wv = attrv[sl]
        m = (lax.iota(jnp.int32, 16) + i * 16) < _EPW
        fi = jnp.where(m, sv * N + dv, 0)
        wv = jnp.where(m, wv, 0.0)
        r = i // 8
        cb = pl.ds((i % 8) * 16, 16)
        aidx[r, cb] = fi
        aval[r, cb] = wv
        cidx[r, cb] = jnp.where(m, dv, 0)
        cval[r, cb] = jnp.where(m, 1.0, 0.0)

    plsc.subcore_barrier()
    for j in range(8):
        pltpu.sync_copy(aval.at[j], a_sh.at[aidx.at[j]], add=True)
        pltpu.sync_copy(cval.at[j], c_sh.at[cidx.at[j]], add=True)
    plsc.subcore_barrier()

    dst_off = pl.multiple_of(c * _FLAT + s * 40000, 8)
    pltpu.sync_copy(a_sh.at[pl.ds(pl.multiple_of(s * 40000, 8), 40000)], vbounce)
    pltpu.sync_copy(vbounce, a_out.at[pl.ds(dst_off, 40000)])

    @pl.when(s == 1)
    def _wc():
        pltpu.sync_copy(c_sh.at[pl.ds(0, 800)], zbuf.at[pl.ds(0, 800)])
        pltpu.sync_copy(zbuf.at[pl.ds(0, 800)],
                        c_out.at[pl.ds(pl.multiple_of(c * N, 8), 800)])


@jax.jit
def _sc_scatter(srcp, dstp, attrp):
    f32 = jnp.float32
    kfn = pl.kernel(
        _sc_body,
        mesh=plsc.VectorSubcoreMesh(core_axis_name="c", subcore_axis_name="s"),
        out_type=[jax.ShapeDtypeStruct((2 * _FLAT,), f32),
                  jax.ShapeDtypeStruct((2 * N,), f32)],
        scratch_types=[
            pltpu.VMEM((_EPAD,), jnp.int32),
            pltpu.VMEM((_EPAD,), jnp.int32),
            pltpu.VMEM((_EPAD,), f32),
            pltpu.VMEM((8, 128), jnp.int32),
            pltpu.VMEM((8, 128), f32),
            pltpu.VMEM((8, 128), jnp.int32),
            pltpu.VMEM((8, 128), f32),
            pltpu.VMEM((8000,), f32),
            pltpu.VMEM((40000,), f32),
            pltpu.VMEM_SHARED((_FLAT,), f32),
            pltpu.VMEM_SHARED((N,), f32),
        ],
    )
    return kfn(srcp, dstp, attrp)


def _dot(a, b, ca, cb):
    return lax.dot_general(a, b, (((ca,), (cb,)), ((), ())),
                           precision=lax.Precision.HIGHEST,
                           preferred_element_type=jnp.float32)


def _bdot(a, b, ca, cb):
    return lax.dot_general(a.astype(jnp.bfloat16), b.astype(jnp.bfloat16),
                           (((ca,), (cb,)), ((), ())),
                           preferred_element_type=jnp.float32)


def _bf(x):
    return x.astype(jnp.bfloat16).astype(jnp.float32)


def _sigmoid(x):
    return 1.0 / (1.0 + jnp.exp(-x))


def _tc_body(x_ref, pos_ref, a0a_ref, a0b_ref, cnt0_ref, cnt1_ref,
             w1cat_ref, n1w1_ref, p1w_ref, w2cat_ref, n2w1_ref, p2w_ref,
             fc1w_ref, fc1b_ref, bn1g_ref, bn1b_ref,
             fc2w_ref, fc2b_ref, bn2g_ref, bn2b_ref,
             hw0_ref, hw1_ref, hb_ref,
             out_ref, s1o_ref, s2o_ref, p1o_ref, p2o_ref):
    f32 = jnp.float32
    x = x_ref[...]
    pos = pos_ref[...]
    A0 = a0a_ref[...] + a0b_ref[...]
    cnt_row = cnt0_ref[...] + cnt1_ref[...]

    i0 = lax.broadcasted_iota(jnp.int32, (N, N), 0)
    i1 = lax.broadcasted_iota(jnp.int32, (N, N), 1)
    eye = jnp.where(i0 == i1, 1.0, 0.0).astype(f32)
    same = (i0 // R) == (i1 // R)
    arange_col = lax.broadcasted_iota(jnp.int32, (N, 1), 0).astype(f32)
    i0f = i0.astype(f32)
    goff_row = ((lax.broadcasted_iota(jnp.int32, (1, N), 1) // R) * R).astype(f32)

    def col_of(row):
        return jnp.transpose(row, (1, 0))

    def row_of(col):
        return jnp.transpose(col, (1, 0))

    h = _bf(jnp.maximum(_bdot(pos, n1w1_ref[...], 1, 0), 0.0))
    G = _dot(x, w1cat_ref[...], 1, 0)
    xt = G[:, K * D1:(K + 1) * D1]
    for k in range(K):
        xt = xt + h[:, k:k + 1] * G[:, k * D1:(k + 1) * D1]

    agg = _dot(A0, xt, 0, 0)
    cnt_col = col_of(jnp.maximum(cnt_row, 1.0))
    h1 = agg / cnt_col

    p1w = p1w_ref[...]
    nrm1 = jnp.sqrt(jnp.sum(p1w * p1w, keepdims=True))
    s1_col = _sigmoid(_bdot(h1, p1w, 1, 0) / nrm1)
    s1_row = row_of(s1_col)

    Dm = same & ((s1_col > s1_row) | ((s1_col == s1_row) & (i0 < i1)))
    rank1_row = jnp.sum(jnp.where(Dm, 1.0, 0.0), axis=0, keepdims=True)
    inv1_row = rank1_row + goff_row
    inv1_col = col_of(inv1_row)
    P1 = jnp.where(inv1_row == i0f, 1.0, 0.0)
    perm1_col = _dot(P1, arange_col, 1, 0)
    s1_sorted = _dot(P1, s1_col, 1, 0)
    p1o_ref[...] = perm1_col.astype(jnp.int32)
    s1o_ref[...] = _sigmoid(s1_sorted)

    h1s = h1 * s1_col
    xp = _dot(P1, h1s, 1, 0)
    posp = _dot(P1, pos, 1, 0)

    parts = []
    for g in range(B):
        blk = xp[g * R:(g + 1) * R]
        parts.append(jnp.concatenate(
            [jnp.max(blk, axis=0, keepdims=True),
             jnp.sum(blk, axis=0, keepdims=True) / float(R)], axis=1))
    x1 = jnp.concatenate(parts, axis=0)

    Ap = _dot(_dot(P1, A0, 1, 0), P1, 1, 1)
    Bp = Ap + eye
    M0 = _bdot(Bp, Bp, 1, 0)
    M0 = M0 * (1.0 - eye)

    h2m = _bf(jnp.maximum(_bdot(posp, n2w1_ref[...], 1, 0), 0.0))
    G2 = _dot(xp, w2cat_ref[...], 1, 0)
    xt2 = G2[:, K * D2:(K + 1) * D2]
    for k in range(K):
        xt2 = xt2 + h2m[:, k:k + 1] * G2[:, k * D2:(k + 1) * D2]

    nz_row = jnp.sum(jnp.where(M0 != 0.0, 1.0, 0.0), axis=0, keepdims=True)
    cnt2_col = col_of(jnp.maximum(nz_row, 1.0))
    h2 = _bdot(M0, xt2, 0, 0) / cnt2_col

    p2w = p2w_ref[...]
    nrm2 = jnp.sqrt(jnp.sum(p2w * p2w, keepdims=True))
    s2_col = _sigmoid(_bdot(h2, p2w, 1, 0) / nrm2)
    s2_row = row_of(s2_col)

    D2m = same & ((s2_col > s2_row) | ((s2_col == s2_row) & (i0 < i1)))
    rank2_row = jnp.sum(jnp.where(D2m, 1.0, 0.0), axis=0, keepdims=True)
    rank2_row = rank2_row + goff_row
    P2 = jnp.where(rank2_row == i0f, 1.0, 0.0)
    perm2_col = _dot(P2, arange_col, 1, 0)
    s2_sorted = _dot(P2, s2_col, 1, 0)
    p2o_ref[...] = perm2_col.astype(jnp.int32)
    s2o_ref[...] = _sigmoid(s2_sorted)

    h2s = h2 * s2_col
    parts = []
    for g in range(B):
        blk = h2s[g * R:(g + 1) * R]
        parts.append(jnp.concatenate(
            [jnp.max(blk, axis=0, keepdims=True),
             jnp.sum(blk, axis=0, keepdims=True) / float(R)], axis=1))
    x2 = jnp.concatenate(parts, axis=0)

    xc = jnp.concatenate([x1, x2], axis=1)
    hh = jnp.maximum(_bdot(xc, fc1w_ref[...], 1, 0) + fc1b_ref[...], 0.0)

    def bn(v, g_, b_):
        m = jnp.sum(v, axis=0, keepdims=True) / float(B)
        d = v - m
        var = jnp.sum(d * d, axis=0, keepdims=True) / float(B)
        return d / jnp.sqrt(var + 1e-5) * g_ + b_

    hh = bn(hh, bn1g_ref[...], bn1b_ref[...])
    z = _bdot(hh, fc2w_ref[...], 1, 0) + fc2b_ref[...]
    zm = z - jnp.max(z, axis=1, keepdims=True)
    ls = zm - jnp.log(jnp.sum(jnp.exp(zm), axis=1, keepdims=True))
    hh = bn(ls, bn2g_ref[...], bn2b_ref[...])

    hb = hb_ref[...]
    out_ref[0:B, :] = _bdot(hh, hw0_ref[...], 1, 0) + hb[0:1, :]
    out_ref[B:2 * B, :] = _bdot(hh, hw1_ref[...], 1, 0) + hb[1:2, :]


@jax.jit
def _tc_main(args):
    f32 = jnp.float32
    outs = [
        jax.ShapeDtypeStruct((2 * B, D2), f32),
        jax.ShapeDtypeStruct((N, 1), f32),
        jax.ShapeDtypeStruct((N, 1), f32),
        jax.ShapeDtypeStruct((N, 1), jnp.int32),
        jax.ShapeDtypeStruct((N, 1), jnp.int32),
    ]
    return pl.pallas_call(_tc_body, out_shape=outs)(*args)


def kernel(x, edge_index, batch, edge_attr, pos, n1_w1, n1_w2, n1_b2, pool1_w,
           n2_w1, n2_w2, n2_b2, pool2_w, fc1_w, fc1_b, bn1_g, bn1_b,
           fc2_w, fc2_b, bn2_g, bn2_b, heads_w, heads_b):
    f32 = jnp.float32
    pad = _NW * _EPAD - E
    srcp = jnp.concatenate([edge_index[0], jnp.zeros((pad,), jnp.int32)])
    dstp = jnp.concatenate([edge_index[1], jnp.zeros((pad,), jnp.int32)])
    attrp = jnp.concatenate([edge_attr, jnp.zeros((pad,), f32)])

    a_flat, c_flat = _sc_scatter(srcp, dstp, attrp)
    a0a = a_flat[:_FLAT].reshape(N, N)
    a0b = a_flat[_FLAT:].reshape(N, N)
    cnt0 = c_flat[:N].reshape(1, N)
    cnt1 = c_flat[N:].reshape(1, N)

    rbf = lambda a: a.astype(jnp.bfloat16).astype(f32)
    w1cat = jnp.concatenate(
        [rbf(n1_w2.reshape(K, INDIM, D1).transpose(1, 0, 2).reshape(INDIM, K * D1)),
         n1_b2.reshape(INDIM, D1)], axis=1)
    w2cat = jnp.concatenate(
        [rbf(n2_w2.reshape(K, D1, D2).transpose(1, 0, 2).reshape(D1, K * D2)),
         n2_b2.reshape(D1, D2)], axis=1)

    args = (x, pos, a0a, a0b, cnt0, cnt1,
            w1cat, n1_w1, pool1_w.reshape(D1, 1), w2cat, n2_w1,
            pool2_w.reshape(D2, 1),
            fc1_w, fc1_b.reshape(1, D2), bn1_g.reshape(1, D2),
            bn1_b.reshape(1, D2),
            fc2_w, fc2_b.reshape(1, NCLASS), bn2_g.reshape(1, NCLASS),
            bn2_b.reshape(1, NCLASS),
            heads_w[0], heads_w[1], heads_b)

    out_s, s1o, s2o, p1o, p2o = _tc_main(args)
    outputs = out_s.reshape(2, B, D2)
    return (outputs, pool1_w, pool2_w,
            s1o.reshape(B, R), s2o.reshape(B, R),
            p1o.reshape(N), p2o.reshape(N))
